# baseline (speedup 1.0000x reference)
import functools

import jax
import jax.numpy as jnp
from jax import lax
from jax.experimental import pallas as pl
from jax.experimental.pallas import tpu as pltpu
from jax.experimental.pallas import tpu_sc as plsc

N_NODES = 10000
N_EDGES = 320000
D_FEAT = 128

NUM_CORES = 2
NUM_SUBCORES = 16
NUM_WORKERS = NUM_CORES * NUM_SUBCORES
EDGES_PER_WORKER = N_EDGES // NUM_WORKERS
LANES = 16

ROW_BLOCK = 2048
N_BLOCKS = 5
CNT_PAD = N_BLOCKS * ROW_BLOCK


EDGE_TILES = N_EDGES // 128
MAX_WORKER_TILES = 79


def _histogram_body(edges_hbm, out_hbm, idx_v, cnt_v, sem):
    c = lax.axis_index("c")
    s = lax.axis_index("s")
    wid = s * NUM_CORES + c
    t0 = (wid * EDGE_TILES) // NUM_WORKERS
    t1 = ((wid + 1) * EDGE_TILES) // NUM_WORKERS

    cp = pltpu.async_copy(
        edges_hbm.at[:, pl.ds(t0 * 128, MAX_WORKER_TILES * 128)], idx_v, sem
    )

    zeros = jnp.zeros((LANES,), jnp.float32)

    def zero_body(i, _):
        cnt_v[pl.ds(i * LANES, LANES)] = zeros
        return ()

    lax.fori_loop(0, CNT_PAD // LANES, zero_body, (), unroll=8)

    cp.wait()

    ones = jnp.ones((LANES,), jnp.float32)

    @plsc.parallel_loop(0, t1 - t0, 1, unroll=4)
    def tile_body(t):
        for j in range(128 // LANES):
            idx = idx_v[1, pl.ds(t * 128 + j * LANES, LANES)]
            plsc.addupdate_scatter(cnt_v, [idx], ones)

    pltpu.sync_copy(cnt_v, out_hbm.at[wid])


@functools.cache
def _histogram():
    return pl.kernel(
        _histogram_body,
        out_type=jax.ShapeDtypeStruct((NUM_WORKERS, CNT_PAD), jnp.float32),
        mesh=plsc.VectorSubcoreMesh(core_axis_name="c", subcore_axis_name="s"),
        scratch_types=[
            pltpu.VMEM((2, MAX_WORKER_TILES * 128), jnp.int32),
            pltpu.VMEM((CNT_PAD,), jnp.float32),
            pltpu.SemaphoreType.DMA,
        ],
        compiler_params=pltpu.CompilerParams(
            needs_layout_passes=False,
            skip_device_barrier=True,
            disable_bounds_checks=True,
            disable_semaphore_checks=True,
        ),
        name="edge_target_histogram",
    )


def _scale_matmul_body(x_ref, w_ref, cnt_ref, o_ref):
    deg = jnp.sum(cnt_ref[...], axis=0)
    y = jnp.dot(x_ref[...], w_ref[...], preferred_element_type=jnp.float32)
    o_ref[...] = y * deg[:, None]


def kernel(edge_index, x, W):
    partial_counts = _histogram()(edge_index)

    out = pl.pallas_call(
        _scale_matmul_body,
        grid=(N_BLOCKS,),
        in_specs=[
            pl.BlockSpec((ROW_BLOCK, D_FEAT), lambda i: (i, 0)),
            pl.BlockSpec((D_FEAT, D_FEAT), lambda i: (0, 0)),
            pl.BlockSpec((NUM_WORKERS, ROW_BLOCK), lambda i: (0, i)),
        ],
        out_specs=pl.BlockSpec((ROW_BLOCK, D_FEAT), lambda i: (i, 0)),
        out_shape=jax.ShapeDtypeStruct((N_NODES, D_FEAT), jnp.float32),
    )(x, W, partial_counts)
    return out


# speedup vs baseline: 1.0007x; 1.0007x over previous
import functools

import jax
import jax.numpy as jnp
from jax import lax
from jax.experimental import pallas as pl
from jax.experimental.pallas import tpu as pltpu
from jax.experimental.pallas import tpu_sc as plsc

N_NODES = 10000
N_EDGES = 320000
D_FEAT = 128

NUM_CORES = 2
NUM_SUBCORES = 16
NUM_WORKERS = NUM_CORES * NUM_SUBCORES
EDGES_PER_WORKER = N_EDGES // NUM_WORKERS
LANES = 16

ROW_BLOCK = 2048
N_BLOCKS = 5
CNT_PAD = N_BLOCKS * ROW_BLOCK


EDGE_TILES = N_EDGES // 128
MAX_WORKER_TILES = 79


def _histogram_body(edges_hbm, out_hbm, idx_v, cnt_v, sem):
    c = lax.axis_index("c")
    s = lax.axis_index("s")
    wid = s * NUM_CORES + c
    t0 = (wid * EDGE_TILES) // NUM_WORKERS
    t1 = ((wid + 1) * EDGE_TILES) // NUM_WORKERS

    cp = pltpu.async_copy(
        edges_hbm.at[:, pl.ds(t0 * 128, MAX_WORKER_TILES * 128)], idx_v, sem
    )

    zeros = jnp.zeros((LANES,), jnp.float32)

    def zero_body(i, _):
        cnt_v[pl.ds(i * LANES, LANES)] = zeros
        return ()

    lax.fori_loop(0, CNT_PAD // LANES, zero_body, (), unroll=8)

    cp.wait()

    ones = jnp.ones((LANES,), jnp.float32)

    @plsc.parallel_loop(0, t1 - t0, 1, unroll=2)
    def tile_body(t):
        for j in range(128 // LANES):
            idx = idx_v[1, pl.ds(t * 128 + j * LANES, LANES)]
            plsc.addupdate_scatter(cnt_v, [idx], ones)

    pltpu.sync_copy(cnt_v, out_hbm.at[wid])


@functools.cache
def _histogram():
    return pl.kernel(
        _histogram_body,
        out_type=jax.ShapeDtypeStruct((NUM_WORKERS, CNT_PAD), jnp.float32),
        mesh=plsc.VectorSubcoreMesh(core_axis_name="c", subcore_axis_name="s"),
        scratch_types=[
            pltpu.VMEM((2, MAX_WORKER_TILES * 128), jnp.int32),
            pltpu.VMEM((CNT_PAD,), jnp.float32),
            pltpu.SemaphoreType.DMA,
        ],
        compiler_params=pltpu.CompilerParams(
            needs_layout_passes=False,
            skip_device_barrier=True,
            disable_bounds_checks=True,
            disable_semaphore_checks=True,
        ),
        name="edge_target_histogram",
    )


def _matmul_body(x_ref, w_ref, y_ref):
    y = jnp.dot(x_ref[...], w_ref[...], preferred_element_type=jnp.float32)
    y_ref[...] = y.astype(jnp.bfloat16)


def _scale_body(y_ref, cnt_ref, o_ref):
    deg = jnp.sum(cnt_ref[...], axis=0)
    o_ref[...] = y_ref[...].astype(jnp.float32) * deg[:, None]


def kernel(edge_index, x, W):
    partial_counts = _histogram()(edge_index)

    y = pl.pallas_call(
        _matmul_body,
        grid=(N_BLOCKS,),
        in_specs=[
            pl.BlockSpec((ROW_BLOCK, D_FEAT), lambda i: (i, 0)),
            pl.BlockSpec((D_FEAT, D_FEAT), lambda i: (0, 0)),
        ],
        out_specs=pl.BlockSpec((ROW_BLOCK, D_FEAT), lambda i: (i, 0)),
        out_shape=jax.ShapeDtypeStruct((N_NODES, D_FEAT), jnp.bfloat16),
    )(x, W)

    out = pl.pallas_call(
        _scale_body,
        grid=(N_BLOCKS,),
        in_specs=[
            pl.BlockSpec((ROW_BLOCK, D_FEAT), lambda i: (i, 0)),
            pl.BlockSpec((NUM_WORKERS, ROW_BLOCK), lambda i: (0, i)),
        ],
        out_specs=pl.BlockSpec((ROW_BLOCK, D_FEAT), lambda i: (i, 0)),
        out_shape=jax.ShapeDtypeStruct((N_NODES, D_FEAT), jnp.float32),
    )(y, partial_counts)
    return out


# speedup vs baseline: 1.0076x; 1.0069x over previous
import functools

import jax
import jax.numpy as jnp
from jax import lax
from jax.experimental import pallas as pl
from jax.experimental.pallas import tpu as pltpu
from jax.experimental.pallas import tpu_sc as plsc

N_NODES = 10000
N_EDGES = 320000
D_FEAT = 128

NUM_CORES = 2
NUM_SUBCORES = 16
NUM_WORKERS = NUM_CORES * NUM_SUBCORES
EDGES_PER_WORKER = N_EDGES // NUM_WORKERS
LANES = 16

ROW_BLOCK = 2048
N_BLOCKS = 5
CNT_PAD = N_BLOCKS * ROW_BLOCK


EDGE_TILES = N_EDGES // 128
MAX_WORKER_TILES = 79


def _histogram_body(edges_hbm, out_hbm, idx_v, cnt_v, sem):
    c = lax.axis_index("c")
    s = lax.axis_index("s")
    wid = s * NUM_CORES + c
    t0 = (wid * EDGE_TILES) // NUM_WORKERS
    t1 = ((wid + 1) * EDGE_TILES) // NUM_WORKERS

    cp = pltpu.async_copy(
        edges_hbm.at[:, pl.ds(t0 * 128, MAX_WORKER_TILES * 128)], idx_v, sem
    )

    zeros = jnp.zeros((LANES,), jnp.float32)

    def zero_body(i, _):
        cnt_v[pl.ds(i * LANES, LANES)] = zeros
        return ()

    lax.fori_loop(0, CNT_PAD // LANES, zero_body, (), unroll=8)

    cp.wait()

    ones = jnp.ones((LANES,), jnp.float32)

    @plsc.parallel_loop(0, t1 - t0, 1, unroll=2)
    def tile_body(t):
        for j in range(128 // LANES):
            idx = idx_v[1, pl.ds(t * 128 + j * LANES, LANES)]
            plsc.addupdate_scatter(cnt_v, [idx], ones)

    pltpu.sync_copy(cnt_v, out_hbm.at[wid])


@functools.cache
def _histogram():
    return pl.kernel(
        _histogram_body,
        out_type=jax.ShapeDtypeStruct((NUM_WORKERS, CNT_PAD), jnp.float32),
        mesh=plsc.VectorSubcoreMesh(core_axis_name="c", subcore_axis_name="s"),
        scratch_types=[
            pltpu.VMEM((2, MAX_WORKER_TILES * 128), jnp.int32),
            pltpu.VMEM((CNT_PAD,), jnp.float32),
            pltpu.SemaphoreType.DMA,
        ],
        compiler_params=pltpu.CompilerParams(
            needs_layout_passes=False,
            skip_device_barrier=True,
            disable_bounds_checks=True,
            disable_semaphore_checks=True,
        ),
        name="edge_target_histogram",
    )


def _scale_matmul_body(x_ref, w_ref, cnt_ref, o_ref):
    deg = jnp.sum(cnt_ref[...], axis=0)
    y = jnp.dot(x_ref[...], w_ref[...], preferred_element_type=jnp.float32)
    o_ref[...] = y * deg[:, None]


def kernel(edge_index, x, W):
    partial_counts = _histogram()(edge_index)

    out = pl.pallas_call(
        _scale_matmul_body,
        grid=(N_BLOCKS,),
        in_specs=[
            pl.BlockSpec((ROW_BLOCK, D_FEAT), lambda i: (i, 0)),
            pl.BlockSpec((D_FEAT, D_FEAT), lambda i: (0, 0)),
            pl.BlockSpec((NUM_WORKERS, ROW_BLOCK), lambda i: (0, i)),
        ],
        out_specs=pl.BlockSpec((ROW_BLOCK, D_FEAT), lambda i: (i, 0)),
        out_shape=jax.ShapeDtypeStruct((N_NODES, D_FEAT), jnp.float32),
    )(x, W, partial_counts)
    return out
